# baseline (speedup 1.0000x reference)
import jax
import jax.numpy as jnp
from jax import lax
from jax.experimental import pallas as pl
from jax.experimental.pallas import tpu as pltpu
from jax.experimental.pallas import tpu_sc as plsc

N_USERS = 40000
N_ITEMS = 60000
N_NODES = N_USERS + N_ITEMS
D = 16
HD = 8
E = 3200000
NC, NS = 2, 16
NW = NC * NS
R = 7
CE = R * 128
NCHUNK = 225
RPT = R * NCHUNK
EPT = RPT * 128
E_PAD = EPT * NS
NPT = N_NODES // NS
BATCH = 16384
BPW = BATCH // NW

_mesh = plsc.VectorSubcoreMesh(core_axis_name="c", subcore_axis_name="s")
_params = pltpu.CompilerParams(
    use_tc_tiling_on_sc=False, needs_layout_passes=False
)


def _prop_body(emb, zt, src2, dst2, w2, hist, bufX, bufY,
               src0, dst0, w0, rows0, src1, dst1, w1, rows1,
               src2b, dst2b, w2b, rows2b,
               gs0, gs1, gs2, ss0, ss1, ss2, is0, is1, is2):
    cid = lax.axis_index("c")
    sid = lax.axis_index("s")
    nsl = pl.ds(sid * NPT, NPT)

    pltpu.sync_copy(emb.at[nsl, pl.ds(cid * HD, HD)], bufX.at[nsl])
    pltpu.sync_copy(bufX.at[nsl], hist.at[3].at[cid].at[nsl])

    S0 = (src0, dst0, w0, rows0, gs0, ss0, is0)
    S1 = (src1, dst1, w1, rows1, gs1, ss1, is1)
    S2 = (src2b, dst2b, w2b, rows2b, gs2, ss2, is2)
    base = sid * RPT
    lanes = jnp.arange(16, dtype=jnp.int32)
    lhalf = jnp.where(lanes < 8, 0, 1)
    cols = jnp.bitwise_and(lanes, 7)
    _pats = [
        jnp.where(lanes < 8, 2 * k, 2 * k + 1).astype(jnp.int32)
        for k in range(8)
    ]

    def fetch_idx(S, ci):
        sv, dv, wv, _, _, _, isem = S
        r0 = base + ci * R
        pltpu.async_copy(src2.at[pl.ds(r0, R)], sv, isem)
        pltpu.async_copy(dst2.at[pl.ds(r0, R)], dv, isem)
        pltpu.async_copy(w2.at[pl.ds(r0, R)], wv, isem)

    def drain_idx(S):
        sv, dv, wv, _, _, _, isem = S
        pltpu.make_async_copy(src2.at[pl.ds(0, R)], sv, isem).wait()
        pltpu.make_async_copy(dst2.at[pl.ds(0, R)], dv, isem).wait()
        pltpu.make_async_copy(w2.at[pl.ds(0, R)], wv, isem).wait()

    def make_layer(cur, acc):
        def fire_gathers(S):
            sv, _, _, rv, gsem, _, _ = S
            for j in range(R):
                pltpu.async_copy(
                    cur.at[sv.at[j]], rv.at[pl.ds(j * 128, 128)], gsem
                )

        def drain_gathers(S):
            sv, _, _, rv, gsem, _, _ = S
            for j in range(R):
                pltpu.make_async_copy(
                    cur.at[sv.at[j]], rv.at[pl.ds(j * 128, 128)], gsem
                ).wait()

        def scale_scatter(S):
            _, dv, wv, rv, _, ssem, _ = S
            for j in range(R):
                def _sgrp(g, carry3, j=j):
                    w16 = wv[j, pl.ds(g * 16, 16)]

                    def _spair(k2, carry4):
                        for u in range(2):
                            k = 2 * k2 + u
                            i = j * 128 + g * 16 + 2 * k
                            ri = i + lhalf
                            v = plsc.load_gather(rv, [ri, cols])
                            wpair = w16[2 * k + lhalf]
                            plsc.store_scatter(rv, [ri, cols], v * wpair)
                        return carry4

                    return lax.fori_loop(0, 4, _spair, carry3)

                lax.fori_loop(0, 8, _sgrp, None)
                pltpu.async_copy(
                    rv.at[pl.ds(j * 128, 128)], acc.at[dv.at[j]],
                    ssem, add=True
                )

        def wait_scatter(S):
            _, _, _, rv, _, ssem, _ = S
            for j in range(R):
                pltpu.make_async_copy(
                    zt.at[pl.ds(0, 128)], rv.at[pl.ds(j * 128, 128)], ssem
                ).wait()

        def step(Sa, Sb, Sc, n):
            @pl.when(n + 1 < NCHUNK)
            def _():
                drain_idx(Sb)
                fire_gathers(Sb)

            @pl.when(n >= 1)
            def _():
                wait_scatter(Sc)

            @pl.when(n + 2 < NCHUNK)
            def _():
                fetch_idx(Sc, n + 2)

            drain_gathers(Sa)
            scale_scatter(Sa)

        def run():
            pltpu.sync_copy(zt.at[nsl], acc.at[nsl])
            plsc.subcore_barrier()
            fetch_idx(S0, 0)
            drain_idx(S0)
            fire_gathers(S0)
            fetch_idx(S1, 1)

            def _trip(g, carry):
                n0 = 3 * g
                step(S0, S1, S2, n0)
                step(S1, S2, S0, n0 + 1)
                step(S2, S0, S1, n0 + 2)
                return carry

            lax.fori_loop(0, NCHUNK // 3, _trip, None)
            wait_scatter(S2)
            plsc.subcore_barrier()

        return run

    for layer, (cur, acc) in enumerate(((bufX, bufY), (bufY, bufX),
                                        (bufX, bufY))):
        make_layer(cur, acc)()
        pltpu.sync_copy(acc.at[nsl], hist.at[layer].at[cid].at[nsl])


def _gamma_body(t0, t1, t2, t3, t4, t5, t6, t7, users, items, gamma_out,
                uidx, iidx, uh0, uh1, ih0, ih1, ov, gsem):
    cid = lax.axis_index("c")
    sid = lax.axis_index("s")
    wid = cid * NS + sid
    b0 = wid * BPW
    pltpu.sync_copy(users.at[pl.ds(b0, BPW)], uidx)
    pltpu.sync_copy(items.at[pl.ds(b0, BPW)], iidx)
    cps = []
    for t in range(BPW // 128):
        sl = pl.ds(t * 128, 128)
        dsl = pl.ds(t * 128, 128)
        for tab, dstb, idx in (
            (t0, uh0, uidx), (t4, uh1, uidx), (t0, ih0, iidx), (t4, ih1, iidx),
        ):
            cps.append(
                pltpu.async_copy(tab.at[idx.at[sl]], dstb.at[dsl], gsem)
            )
    for cp in cps:
        cp.wait()
    cps = []
    for t in range(BPW // 128):
        sl = pl.ds(t * 128, 128)
        dsl = pl.ds(t * 128, 128)
        for tab, dstb, idx in (
            (t1, uh0, uidx), (t2, uh0, uidx), (t3, uh0, uidx),
            (t5, uh1, uidx), (t6, uh1, uidx), (t7, uh1, uidx),
            (t1, ih0, iidx), (t2, ih0, iidx), (t3, ih0, iidx),
            (t5, ih1, iidx), (t6, ih1, iidx), (t7, ih1, iidx),
        ):
            cps.append(
                pltpu.async_copy(
                    tab.at[idx.at[sl]], dstb.at[dsl], gsem, add=True
                )
            )
    for cp in cps:
        cp.wait()

    lanes = jnp.arange(16, dtype=jnp.int32)
    lo = lanes < 8

    def _dot(g, carry):
        acc = jnp.zeros((16,), jnp.float32)
        for k in range(8):
            r = 2 * (g * 8 + k)
            ri = r + jnp.where(lo, 0, 1)
            cols = jnp.bitwise_and(lanes, 7)
            prod = (
                plsc.load_gather(uh0, [ri, cols])
                * plsc.load_gather(ih0, [ri, cols])
                + plsc.load_gather(uh1, [ri, cols])
                * plsc.load_gather(ih1, [ri, cols])
            )
            s0 = jnp.sum(jnp.where(lo, prod, 0.0))
            s1 = jnp.sum(jnp.where(lo, 0.0, prod))
            acc = jnp.where(lanes == 2 * k, s0, acc)
            acc = jnp.where(lanes == 2 * k + 1, s1, acc)
        ov[pl.ds(g * 16, 16)] = acc * (1.0 / 16.0)
        return carry

    lax.fori_loop(0, BPW // 16, _dot, None)
    pltpu.sync_copy(ov, gamma_out.at[pl.ds(b0, BPW)])


_prop = pl.kernel(
    _prop_body,
    out_type=jax.ShapeDtypeStruct((4, NC, N_NODES, HD), jnp.float32),
    mesh=_mesh,
    compiler_params=_params,
    scratch_types=(
        [
            pltpu.VMEM_SHARED((N_NODES, HD), jnp.float32),
            pltpu.VMEM_SHARED((N_NODES, HD), jnp.float32),
        ]
        + [
            pltpu.VMEM((R, 128), jnp.int32),
            pltpu.VMEM((R, 128), jnp.int32),
            pltpu.VMEM((R, 128), jnp.float32),
            pltpu.VMEM((CE, HD), jnp.float32),
        ] * 3
        + [pltpu.SemaphoreType.DMA] * 9
    ),
)

_gamma = pl.kernel(
    _gamma_body,
    out_type=jax.ShapeDtypeStruct((BATCH,), jnp.float32),
    mesh=_mesh,
    compiler_params=_params,
    scratch_types=(
        [pltpu.VMEM((BPW,), jnp.int32)] * 2
        + [pltpu.VMEM((BPW, HD), jnp.float32)] * 4
        + [pltpu.VMEM((BPW,), jnp.float32), pltpu.SemaphoreType.DMA]
    ),
)


def kernel(user_emb, item_emb, edge_weight, edge_index, users, items):
    all_emb = jnp.concatenate([user_emb, item_emb], axis=0)
    zt = jnp.zeros((N_NODES, HD), jnp.float32)
    pad = E_PAD - E
    src2 = jnp.concatenate(
        [edge_index[0], jnp.zeros((pad,), jnp.int32)]).reshape(-1, 128)
    dst2 = jnp.concatenate(
        [edge_index[1], jnp.zeros((pad,), jnp.int32)]).reshape(-1, 128)
    w2 = jnp.concatenate(
        [edge_weight, jnp.zeros((pad,), jnp.float32)]).reshape(-1, 128)
    items_g = items + N_USERS

    hist = _prop(all_emb, zt, src2, dst2, w2)
    return _gamma(
        hist[3, 0], hist[0, 0], hist[1, 0], hist[2, 0],
        hist[3, 1], hist[0, 1], hist[1, 1], hist[2, 1],
        users, items_g,
    )


# speedup vs baseline: 1.3837x; 1.3837x over previous
import jax
import jax.numpy as jnp
from jax import lax
from jax.experimental import pallas as pl
from jax.experimental.pallas import tpu as pltpu
from jax.experimental.pallas import tpu_sc as plsc

N_USERS = 40000
N_ITEMS = 60000
N_NODES = N_USERS + N_ITEMS
D = 16
HD = 8
E = 3200000
NC, NS = 2, 16
NW = NC * NS
R = 7
CE = R * 128
NCHUNK = 225
RPT = R * NCHUNK
EPT = RPT * 128
E_PAD = EPT * NS
NPT = N_NODES // NS
BATCH = 16384
BPW = BATCH // NW

_mesh = plsc.VectorSubcoreMesh(core_axis_name="c", subcore_axis_name="s")
_params = pltpu.CompilerParams(
    use_tc_tiling_on_sc=False, needs_layout_passes=False
)


def _prop_body(embT, zt, src2, dst2, w2, hist, bufX, bufY,
               src0, dst0, w0, rows0, src1, dst1, w1, rows1,
               src2b, dst2b, w2b, rows2b,
               gs0, gs1, gs2, ss0, ss1, ss2, is0, is1, is2):
    cid = lax.axis_index("c")
    sid = lax.axis_index("s")
    nsl = pl.ds(sid * NPT, NPT)

    pltpu.sync_copy(embT.at[cid].at[nsl], bufX.at[nsl])
    pltpu.sync_copy(bufX.at[nsl], hist.at[3].at[cid].at[nsl])

    S0 = (src0, dst0, w0, rows0, gs0, ss0, is0)
    S1 = (src1, dst1, w1, rows1, gs1, ss1, is1)
    S2 = (src2b, dst2b, w2b, rows2b, gs2, ss2, is2)
    base = sid * RPT
    lanes = jnp.arange(16, dtype=jnp.int32)
    lhalf = jnp.where(lanes < 8, 0, 1)
    cols = jnp.bitwise_and(lanes, 7)
    _pats = [
        jnp.where(lanes < 8, 2 * k, 2 * k + 1).astype(jnp.int32)
        for k in range(8)
    ]

    def fetch_idx(S, ci):
        sv, dv, wv, _, _, _, isem = S
        r0 = base + ci * R
        pltpu.async_copy(src2.at[pl.ds(r0, R)], sv, isem)
        pltpu.async_copy(dst2.at[pl.ds(r0, R)], dv, isem)
        pltpu.async_copy(w2.at[pl.ds(r0, R)], wv, isem)

    def drain_idx(S):
        sv, dv, wv, _, _, _, isem = S
        pltpu.make_async_copy(src2.at[pl.ds(0, R)], sv, isem).wait()
        pltpu.make_async_copy(dst2.at[pl.ds(0, R)], dv, isem).wait()
        pltpu.make_async_copy(w2.at[pl.ds(0, R)], wv, isem).wait()

    def make_layer(cur, acc):
        def fire_gathers(S):
            sv, _, _, rv, gsem, _, _ = S
            for j in range(R):
                pltpu.async_copy(
                    cur.at[sv.at[j]], rv.at[pl.ds(j * 128, 128)], gsem
                )

        def drain_gathers(S):
            sv, _, _, rv, gsem, _, _ = S
            for j in range(R):
                pltpu.make_async_copy(
                    cur.at[sv.at[j]], rv.at[pl.ds(j * 128, 128)], gsem
                ).wait()

        def scale_scatter(S):
            _, dv, wv, rv, _, ssem, _ = S
            for j in range(R):
                def _sgrp(g, carry3, j=j):
                    w16 = wv[j, pl.ds(g * 16, 16)]

                    def _spair(k2, carry4):
                        for u in range(2):
                            k = 2 * k2 + u
                            i = j * 128 + g * 16 + 2 * k
                            ri = i + lhalf
                            v = plsc.load_gather(rv, [ri, cols])
                            wpair = w16[2 * k + lhalf]
                            plsc.store_scatter(rv, [ri, cols], v * wpair)
                        return carry4

                    return lax.fori_loop(0, 4, _spair, carry3)

                lax.fori_loop(0, 8, _sgrp, None)
                pltpu.async_copy(
                    rv.at[pl.ds(j * 128, 128)], acc.at[dv.at[j]],
                    ssem, add=True
                )

        def wait_scatter(S):
            _, _, _, rv, _, ssem, _ = S
            for j in range(R):
                pltpu.make_async_copy(
                    zt.at[pl.ds(0, 128)], rv.at[pl.ds(j * 128, 128)], ssem
                ).wait()

        def step(Sa, Sb, Sc, n):
            @pl.when(n + 1 < NCHUNK)
            def _():
                drain_idx(Sb)
                fire_gathers(Sb)

            @pl.when(n >= 1)
            def _():
                wait_scatter(Sc)

            @pl.when(n + 2 < NCHUNK)
            def _():
                fetch_idx(Sc, n + 2)

            drain_gathers(Sa)
            scale_scatter(Sa)

        def run():
            pltpu.sync_copy(zt.at[nsl], acc.at[nsl])
            plsc.subcore_barrier()
            fetch_idx(S0, 0)
            drain_idx(S0)
            fire_gathers(S0)
            fetch_idx(S1, 1)

            def _trip(g, carry):
                n0 = 3 * g
                step(S0, S1, S2, n0)
                step(S1, S2, S0, n0 + 1)
                step(S2, S0, S1, n0 + 2)
                return carry

            lax.fori_loop(0, NCHUNK // 3, _trip, None)
            wait_scatter(S2)
            plsc.subcore_barrier()

        return run

    for layer, (cur, acc) in enumerate(((bufX, bufY), (bufY, bufX),
                                        (bufX, bufY))):
        make_layer(cur, acc)()
        pltpu.sync_copy(acc.at[nsl], hist.at[layer].at[cid].at[nsl])


def _gamma_body(hist, users, items, gamma_out,
                uidx, iidx, uh0, uh1, ih0, ih1, ov, gsem):
    cid = lax.axis_index("c")
    sid = lax.axis_index("s")
    wid = cid * NS + sid
    b0 = wid * BPW
    pltpu.sync_copy(users.at[pl.ds(b0, BPW)], uidx)
    pltpu.sync_copy(items.at[pl.ds(b0, BPW)], iidx)
    tabs = [hist.at[layer].at[h] for layer in (3, 0, 1, 2) for h in (0, 1)]
    cps = []
    for t in range(BPW // 128):
        sl = pl.ds(t * 128, 128)
        dsl = pl.ds(t * 128, 128)
        for tab, dstb, idx in (
            (tabs[0], uh0, uidx), (tabs[1], uh1, uidx),
            (tabs[0], ih0, iidx), (tabs[1], ih1, iidx),
        ):
            cps.append(
                pltpu.async_copy(tab.at[idx.at[sl]], dstb.at[dsl], gsem)
            )
    for cp in cps:
        cp.wait()
    cps = []
    for t in range(BPW // 128):
        sl = pl.ds(t * 128, 128)
        dsl = pl.ds(t * 128, 128)
        for tab, dstb, idx in (
            (tabs[2], uh0, uidx), (tabs[4], uh0, uidx), (tabs[6], uh0, uidx),
            (tabs[3], uh1, uidx), (tabs[5], uh1, uidx), (tabs[7], uh1, uidx),
            (tabs[2], ih0, iidx), (tabs[4], ih0, iidx), (tabs[6], ih0, iidx),
            (tabs[3], ih1, iidx), (tabs[5], ih1, iidx), (tabs[7], ih1, iidx),
        ):
            cps.append(
                pltpu.async_copy(
                    tab.at[idx.at[sl]], dstb.at[dsl], gsem, add=True
                )
            )
    for cp in cps:
        cp.wait()

    lanes = jnp.arange(16, dtype=jnp.int32)
    lo = lanes < 8

    def _dot(g, carry):
        acc = jnp.zeros((16,), jnp.float32)
        for k in range(8):
            r = 2 * (g * 8 + k)
            ri = r + jnp.where(lo, 0, 1)
            cols = jnp.bitwise_and(lanes, 7)
            prod = (
                plsc.load_gather(uh0, [ri, cols])
                * plsc.load_gather(ih0, [ri, cols])
                + plsc.load_gather(uh1, [ri, cols])
                * plsc.load_gather(ih1, [ri, cols])
            )
            s0 = jnp.sum(jnp.where(lo, prod, 0.0))
            s1 = jnp.sum(jnp.where(lo, 0.0, prod))
            acc = jnp.where(lanes == 2 * k, s0, acc)
            acc = jnp.where(lanes == 2 * k + 1, s1, acc)
        ov[pl.ds(g * 16, 16)] = acc * (1.0 / 16.0)
        return carry

    lax.fori_loop(0, BPW // 16, _dot, None)
    pltpu.sync_copy(ov, gamma_out.at[pl.ds(b0, BPW)])


_prop = pl.kernel(
    _prop_body,
    out_type=jax.ShapeDtypeStruct((4, NC, N_NODES, HD), jnp.float32),
    mesh=_mesh,
    compiler_params=_params,
    scratch_types=(
        [
            pltpu.VMEM_SHARED((N_NODES, HD), jnp.float32),
            pltpu.VMEM_SHARED((N_NODES, HD), jnp.float32),
        ]
        + [
            pltpu.VMEM((R, 128), jnp.int32),
            pltpu.VMEM((R, 128), jnp.int32),
            pltpu.VMEM((R, 128), jnp.float32),
            pltpu.VMEM((CE, HD), jnp.float32),
        ] * 3
        + [pltpu.SemaphoreType.DMA] * 9
    ),
)

_gamma = pl.kernel(
    _gamma_body,
    out_type=jax.ShapeDtypeStruct((BATCH,), jnp.float32),
    mesh=_mesh,
    compiler_params=_params,
    scratch_types=(
        [pltpu.VMEM((BPW,), jnp.int32)] * 2
        + [pltpu.VMEM((BPW, HD), jnp.float32)] * 4
        + [pltpu.VMEM((BPW,), jnp.float32), pltpu.SemaphoreType.DMA]
    ),
)


def kernel(user_emb, item_emb, edge_weight, edge_index, users, items):
    all_emb = jnp.concatenate([user_emb, item_emb], axis=0)
    embT = all_emb.reshape(N_NODES, NC, HD).transpose(1, 0, 2)
    zt = jnp.zeros((N_NODES, HD), jnp.float32)
    pad = E_PAD - E
    src2 = jnp.concatenate(
        [edge_index[0], jnp.zeros((pad,), jnp.int32)]).reshape(-1, 128)
    dst2 = jnp.concatenate(
        [edge_index[1], jnp.zeros((pad,), jnp.int32)]).reshape(-1, 128)
    w2 = jnp.concatenate(
        [edge_weight, jnp.zeros((pad,), jnp.float32)]).reshape(-1, 128)
    items_g = items + N_USERS

    hist = _prop(embT, zt, src2, dst2, w2)
    return _gamma(hist, users, items_g)
